# Initial kernel scaffold; baseline (speedup 1.0000x reference)
import jax
import jax.numpy as jnp
from jax.experimental import pallas as pl


def kernel(emb_mashup, emb_api, emb_mm, emb_aa, graph_vals, mm_vals, aa_vals, diff1_vals, diff2_vals, mashups, pos_apis, neg_apis, graph_rows, graph_cols, mm_rows, mm_cols, aa_rows, aa_cols, diff1_rows, diff1_cols, diff2_rows, diff2_cols):
    raise NotImplementedError("write your pallas kernel here")


# speedup vs baseline: 1.1107x; 1.1107x over previous
import functools
import jax
import jax.numpy as jnp
from jax import lax
from jax.experimental import pallas as pl
from jax.experimental.pallas import tpu as pltpu

N_M = 25000
N_A = 25000
N = N_M + N_A
DIM = 64
LAYERS = 3
TEMP = 0.2
B = 4096
BLK = 256
NB = B // BLK


def _spmm(rows, cols, vals, x, n):
    return jax.ops.segment_sum(vals[:, None] * jnp.take(x, cols, axis=0),
                               rows, num_segments=n)


def _loss_body(mash_c, mash_r, pos_c, pos_r,
               me0, pe0, ne0, me1, pe1, ne1, me2, pe2, ne2,
               loss_o, reg_o, sslm_o, ssla_o, e1_s, e2_s, mask_s):
    total_loss = jnp.float32(0.0)
    total_reg = jnp.float32(0.0)
    for (me_r, pe_r, ne_r) in ((me0, pe0, ne0), (me1, pe1, ne1),
                               (me2, pe2, ne2)):
        me = me_r[...] * 0.25
        pe = pe_r[...] * 0.25
        ne = ne_r[...] * 0.25
        pos_sc = jnp.sum(me * pe, axis=-1)
        neg_sc = jnp.sum(me * ne, axis=-1)
        d = neg_sc - pos_sc
        sp = jnp.maximum(d, 0.0) + jnp.log1p(jnp.exp(-jnp.abs(d)))
        total_loss = total_loss + jnp.sum(sp) * (1.0 / B)
        total_reg = total_reg + (0.5 / B) * (
            jnp.sum(me * me) + jnp.sum(pe * pe) + jnp.sum(ne * ne))

    def _norm(x):
        n = jnp.sqrt(jnp.sum(x * x, axis=1, keepdims=True))
        return x / jnp.maximum(n, 1e-12)

    def _ssl(x1_r, x2_r, idxc_r, idxr_r):
        e1_s[...] = _norm(x1_r[...])
        e2_s[...] = _norm(x2_r[...])
        idxr = idxr_r[...]

        def mask_blk(b, carry):
            rows = pl.ds(b * BLK, BLK)
            idxb = idxc_r[rows, :]
            eq = (idxb == idxr)
            rowpos = (jax.lax.broadcasted_iota(jnp.int32, (BLK, B), 0)
                      + b * BLK)
            colpos = jax.lax.broadcasted_iota(jnp.int32, (BLK, B), 1)
            dup = jnp.max(jnp.where(eq & (colpos < rowpos), 1.0, 0.0),
                          axis=1)
            mask_s[0, rows] = 1.0 - dup
            return carry

        lax.fori_loop(0, NB, mask_blk, 0, unroll=False)
        maskr = mask_s[...]
        k = jnp.sum(maskr)

        def ssl_blk(b, acc):
            rows = pl.ds(b * BLK, BLK)
            d1b = e1_s[rows, :]
            e2b = e2_s[rows, :]
            s = lax.dot_general(d1b, e2_s[...], (((1,), (1,)), ((), ())),
                                preferred_element_type=jnp.float32)
            s = s * (1.0 / TEMP)
            allsum = jnp.sum(jnp.exp(s) * maskr, axis=1)
            posd = jnp.sum(d1b * e2b, axis=1) * (1.0 / TEMP)
            maskb = mask_s[0, rows]
            return acc + jnp.sum(maskb * (posd - jnp.log(allsum)))

        acc = lax.fori_loop(0, NB, ssl_blk, jnp.float32(0.0),
                            unroll=False)
        return -acc / k

    sslm = _ssl(me1, me2, mash_c, mash_r)
    ssla = _ssl(pe1, pe2, pos_c, pos_r)

    loss_o[...] = jnp.reshape(total_loss, (1, 1))
    reg_o[...] = jnp.reshape(total_reg, (1, 1))
    sslm_o[...] = jnp.reshape(sslm, (1, 1))
    ssla_o[...] = jnp.reshape(ssla, (1, 1))


def _loss_call(mash, pos, gathered):
    mash_c = mash.reshape(B, 1)
    mash_r = mash.reshape(1, B)
    pos_c = pos.reshape(B, 1)
    pos_r = pos.reshape(1, B)
    out_shape = [jax.ShapeDtypeStruct((1, 1), jnp.float32)] * 4
    fn = pl.pallas_call(
        _loss_body,
        out_shape=out_shape,
        scratch_shapes=[pltpu.VMEM((B, DIM), jnp.float32),
                        pltpu.VMEM((B, DIM), jnp.float32),
                        pltpu.VMEM((1, B), jnp.float32)],
    )
    return fn(mash_c, mash_r, pos_c, pos_r, *gathered)


def kernel(emb_mashup, emb_api, emb_mm, emb_aa, graph_vals, mm_vals,
           aa_vals, diff1_vals, diff2_vals, mashups, pos_apis, neg_apis,
           graph_rows, graph_cols, mm_rows, mm_cols, aa_rows, aa_cols,
           diff1_rows, diff1_cols, diff2_rows, diff2_cols):
    x0 = jnp.concatenate([emb_mashup, emb_api], axis=0)
    m = _spmm(mm_rows, mm_cols, mm_vals, emb_mm, N_M)
    a = _spmm(aa_rows, aa_cols, aa_vals, emb_aa, N_A)
    pre = jnp.concatenate([m, a], axis=0)

    gathered = []
    ssl_tabs = {}
    for gi, (rows, cols, vals) in enumerate(
            ((graph_rows, graph_cols, graph_vals),
             (diff1_rows, diff1_cols, diff1_vals),
             (diff2_rows, diff2_cols, diff2_vals))):
        x = x0
        acc = x0
        for _ in range(LAYERS):
            y = _spmm(rows, cols, vals, x, N)
            x = (y + pre) * 0.5
            acc = acc + x
        me = jnp.take(acc, mashups, axis=0)
        pe = jnp.take(acc, N_M + pos_apis, axis=0)
        ne = jnp.take(acc, N_M + neg_apis, axis=0)
        gathered.extend([me, pe, ne])

    loss, reg, sslm, ssla = _loss_call(mashups, pos_apis, gathered)
    return (loss[0, 0], reg[0, 0], sslm[0, 0], ssla[0, 0])


# speedup vs baseline: 3.8864x; 3.4990x over previous
import functools
import jax
import jax.numpy as jnp
from jax import lax
from jax.experimental import pallas as pl
from jax.experimental.pallas import tpu as pltpu
from jax.experimental.pallas import tpu_sc as plsc

N_M = 25000
N_A = 25000
N = N_M + N_A
DIM = 64
LAYERS = 3
TEMP = 0.2
B = 4096
BLK = 256
NB = B // BLK

NC = 2
NT = 16
H = 32
W = 256
JW = W // 128
NWIN_G = 196
NWIN_S = 100
EP_G = NT * NWIN_G * W
EP_S = NT * NWIN_S * W
N_P = 50176
RPT = N_P // NT
RCH = 196


def _sc_body(x0, emm, eaa,
             g_cols, g_rows, g_vals,
             mm_cols, mm_rows, mm_vals,
             aa_cols, aa_rows, aa_vals,
             midx, pidx, nidx,
             gath, pre_h, xa_h, xb_h, acc_h,
             cols_v, rows_v, vals_v, g_v, y_v, a_v, idx_v, out_sp, sem):
    c = lax.axis_index("c")
    s = lax.axis_index("s")

    gdn = lax.GatherDimensionNumbers(offset_dims=(),
                                     collapsed_slice_dims=(0,),
                                     start_index_map=(0,))

    def scale_window():
        def body(gi, _):
            v16 = vals_v[gi, :]
            for l in range(16):
                vb = lax.gather(v16, jnp.full((16, 1), l, jnp.int32),
                                dimension_numbers=gdn, slice_sizes=(1,),
                                mode=lax.GatherScatterMode.PROMISE_IN_BOUNDS)
                r = gi * 16 + l
                g_v[r, 0:16] = g_v[r, 0:16] * vb
                g_v[r, 16:32] = g_v[r, 16:32] * vb
            return 0
        lax.fori_loop(0, W // 16, body, 0, unroll=False)

    def spmm_windows(src2d, colsr, rowsr, valsr, nwin):
        def win(w, _):
            gw = s * nwin + w
            pltpu.sync_copy(colsr.at[pl.ds(gw * JW, JW), :], cols_v)
            pltpu.sync_copy(rowsr.at[pl.ds(gw * JW, JW), :], rows_v)
            pltpu.sync_copy(valsr.at[pl.ds(gw * (W // 16), W // 16), :],
                            vals_v)
            cps = [pltpu.async_copy(src2d.at[cols_v.at[j]],
                                    g_v.at[pl.ds(j * 128, 128), :], sem)
                   for j in range(JW)]
            for cp in cps:
                cp.wait()
            scale_window()
            for j in range(JW):
                pltpu.sync_copy(g_v.at[pl.ds(j * 128, 128), :],
                                out_sp.at[rows_v.at[j]], add=True)
            return 0
        lax.fori_loop(0, nwin, win, 0, unroll=False)

    def zero_out_sp():
        def zb(r, _):
            g_v[r, 0:16] = jnp.zeros((16,), jnp.float32)
            g_v[r, 16:32] = jnp.zeros((16,), jnp.float32)
            return 0
        lax.fori_loop(0, RCH, zb, 0, unroll=False)
        for k in range(RPT // RCH):
            pltpu.sync_copy(g_v.at[pl.ds(0, RCH), :],
                            out_sp.at[pl.ds(s * RPT + k * RCH, RCH), :])

    zero_out_sp()
    plsc.subcore_barrier()
    spmm_windows(emm.at[c], mm_cols, mm_rows, mm_vals, NWIN_S)
    spmm_windows(eaa.at[c], aa_cols, aa_rows, aa_vals, NWIN_S)
    plsc.subcore_barrier()
    pltpu.sync_copy(out_sp.at[pl.ds(s * RPT, RPT), :],
                    pre_h.at[c, pl.ds(s * RPT, RPT), :])
    plsc.subcore_barrier()

    def graph_phase(g):
        for l in range(LAYERS):
            src = x0 if l == 0 else (xa_h if l == 1 else xb_h)
            acc_in = x0 if l == 0 else acc_h
            pltpu.sync_copy(pre_h.at[c, pl.ds(s * RPT, RPT), :],
                            out_sp.at[pl.ds(s * RPT, RPT), :])
            plsc.subcore_barrier()
            spmm_windows(src.at[c],
                         g_cols.at[g], g_rows.at[g], g_vals.at[g], NWIN_G)
            plsc.subcore_barrier()
            for k in range(RPT // RCH):
                r0 = s * RPT + k * RCH
                pltpu.sync_copy(out_sp.at[pl.ds(r0, RCH), :], y_v)
                pltpu.sync_copy(acc_in.at[c, pl.ds(r0, RCH), :], a_v)

                def pb(r, _):
                    for h0 in (0, 16):
                        y = y_v[r, h0:h0 + 16] * 0.5
                        a_v[r, h0:h0 + 16] = a_v[r, h0:h0 + 16] + y
                        y_v[r, h0:h0 + 16] = y
                    return 0
                lax.fori_loop(0, RCH, pb, 0, unroll=False)
                if l == 0:
                    pltpu.sync_copy(y_v, xa_h.at[c, pl.ds(r0, RCH), :])
                elif l == 1:
                    pltpu.sync_copy(y_v, xb_h.at[c, pl.ds(r0, RCH), :])
                pltpu.sync_copy(a_v, acc_h.at[c, pl.ds(r0, RCH), :])
            plsc.subcore_barrier()
        for kind, idxr in enumerate((midx, pidx, nidx)):
            for j in range(2):
                j2 = s * 2 + j
                pltpu.sync_copy(idxr.at[j2], idx_v)
                pltpu.async_copy(acc_h.at[c].at[idx_v.at[0]],
                                 g_v.at[pl.ds(0, 128), :], sem).wait()
                pltpu.sync_copy(g_v.at[pl.ds(0, 128), :],
                                gath.at[c, g, kind,
                                        pl.ds(j2 * 128, 128), :])
        plsc.subcore_barrier()

    def gbody(g, _):
        graph_phase(g)
        return 0
    lax.fori_loop(0, 3, gbody, 0, unroll=False)


def _sc_forward(x0, emm, eaa, g_cols, g_rows, g_vals,
                mm_cols, mm_rows, mm_vals, aa_cols, aa_rows, aa_vals,
                midx, pidx, nidx):
    f32 = jnp.float32
    mesh = plsc.VectorSubcoreMesh(core_axis_name="c", subcore_axis_name="s")
    fn = pl.kernel(
        _sc_body,
        mesh=mesh,
        compiler_params=pltpu.CompilerParams(use_tc_tiling_on_sc=False),
        out_type=[
            jax.ShapeDtypeStruct((NC, 3, 3, B, H), f32),
            jax.ShapeDtypeStruct((NC, N_P, H), f32),
            jax.ShapeDtypeStruct((NC, N_P, H), f32),
            jax.ShapeDtypeStruct((NC, N_P, H), f32),
            jax.ShapeDtypeStruct((NC, N_P, H), f32),
        ],
        scratch_types=[
            pltpu.VMEM((JW, 128), jnp.int32),
            pltpu.VMEM((JW, 128), jnp.int32),
            pltpu.VMEM((W // 16, 16), f32),
            pltpu.VMEM((W, H), f32),
            pltpu.VMEM((RCH, H), f32),
            pltpu.VMEM((RCH, H), f32),
            pltpu.VMEM((1, 128), jnp.int32),
            pltpu.VMEM_SHARED((N_P, H), f32),
            pltpu.SemaphoreType.DMA,
        ],
    )
    return fn(x0, emm, eaa, g_cols, g_rows, g_vals,
              mm_cols, mm_rows, mm_vals, aa_cols, aa_rows, aa_vals,
              midx, pidx, nidx)


def _pad_edges(rows, cols, vals, ep, row_off=0):
    e = rows.shape[0]
    pad = ep - e
    rows = jnp.concatenate([rows + row_off,
                            jnp.arange(pad, dtype=rows.dtype) % N])
    cols = jnp.concatenate([cols, jnp.zeros((pad,), cols.dtype)])
    vals = jnp.concatenate([vals, jnp.zeros((pad,), vals.dtype)])
    return (cols.reshape(ep // 128, 128).astype(jnp.int32),
            rows.reshape(ep // 128, 128).astype(jnp.int32),
            vals.reshape(ep // 16, 16))


def _halves(x, pad_to=None):
    h = jnp.stack([x[:, :H], x[:, H:]], axis=0)
    if pad_to is not None and pad_to > h.shape[1]:
        h = jnp.pad(h, ((0, 0), (0, pad_to - h.shape[1]), (0, 0)))
    return h


def _loss_body(mash_c, mash_r, pos_c, pos_r,
               me0, pe0, ne0, me1, pe1, ne1, me2, pe2, ne2,
               loss_o, reg_o, sslm_o, ssla_o, e1_s, e2_s, mask_s):
    total_loss = jnp.float32(0.0)
    total_reg = jnp.float32(0.0)
    for (me_r, pe_r, ne_r) in ((me0, pe0, ne0), (me1, pe1, ne1),
                               (me2, pe2, ne2)):
        me = me_r[...] * 0.25
        pe = pe_r[...] * 0.25
        ne = ne_r[...] * 0.25
        pos_sc = jnp.sum(me * pe, axis=-1)
        neg_sc = jnp.sum(me * ne, axis=-1)
        d = neg_sc - pos_sc
        sp = jnp.maximum(d, 0.0) + jnp.log1p(jnp.exp(-jnp.abs(d)))
        total_loss = total_loss + jnp.sum(sp) * (1.0 / B)
        total_reg = total_reg + (0.5 / B) * (
            jnp.sum(me * me) + jnp.sum(pe * pe) + jnp.sum(ne * ne))

    def _norm(x):
        n = jnp.sqrt(jnp.sum(x * x, axis=1, keepdims=True))
        return x / jnp.maximum(n, 1e-12)

    def _ssl(x1_r, x2_r, idxc_r, idxr_r):
        e1_s[...] = _norm(x1_r[...])
        e2_s[...] = _norm(x2_r[...])
        idxr = idxr_r[...]

        def mask_blk(b, carry):
            rows = pl.ds(b * BLK, BLK)
            idxb = idxc_r[rows, :]
            eq = (idxb == idxr)
            rowpos = (jax.lax.broadcasted_iota(jnp.int32, (BLK, B), 0)
                      + b * BLK)
            colpos = jax.lax.broadcasted_iota(jnp.int32, (BLK, B), 1)
            dup = jnp.max(jnp.where(eq & (colpos < rowpos), 1.0, 0.0),
                          axis=1)
            mask_s[0, rows] = 1.0 - dup
            return carry

        lax.fori_loop(0, NB, mask_blk, 0, unroll=False)
        maskr = mask_s[...]
        k = jnp.sum(maskr)

        def ssl_blk(b, acc):
            rows = pl.ds(b * BLK, BLK)
            d1b = e1_s[rows, :]
            e2b = e2_s[rows, :]
            sm = lax.dot_general(d1b, e2_s[...], (((1,), (1,)), ((), ())),
                                 preferred_element_type=jnp.float32)
            sm = sm * (1.0 / TEMP)
            allsum = jnp.sum(jnp.exp(sm) * maskr, axis=1)
            posd = jnp.sum(d1b * e2b, axis=1) * (1.0 / TEMP)
            maskb = mask_s[0, rows]
            return acc + jnp.sum(maskb * (posd - jnp.log(allsum)))

        acc = lax.fori_loop(0, NB, ssl_blk, jnp.float32(0.0),
                            unroll=False)
        return -acc / k

    sslm = _ssl(me1, me2, mash_c, mash_r)
    ssla = _ssl(pe1, pe2, pos_c, pos_r)

    loss_o[...] = jnp.reshape(total_loss, (1, 1))
    reg_o[...] = jnp.reshape(total_reg, (1, 1))
    sslm_o[...] = jnp.reshape(sslm, (1, 1))
    ssla_o[...] = jnp.reshape(ssla, (1, 1))


def _loss_call(mash, pos, gathered):
    mash_c = mash.reshape(B, 1)
    mash_r = mash.reshape(1, B)
    pos_c = pos.reshape(B, 1)
    pos_r = pos.reshape(1, B)
    out_shape = [jax.ShapeDtypeStruct((1, 1), jnp.float32)] * 4
    fn = pl.pallas_call(
        _loss_body,
        out_shape=out_shape,
        scratch_shapes=[pltpu.VMEM((B, DIM), jnp.float32),
                        pltpu.VMEM((B, DIM), jnp.float32),
                        pltpu.VMEM((1, B), jnp.float32)],
    )
    return fn(mash_c, mash_r, pos_c, pos_r, *gathered)


def kernel(emb_mashup, emb_api, emb_mm, emb_aa, graph_vals, mm_vals,
           aa_vals, diff1_vals, diff2_vals, mashups, pos_apis, neg_apis,
           graph_rows, graph_cols, mm_rows, mm_cols, aa_rows, aa_cols,
           diff1_rows, diff1_cols, diff2_rows, diff2_cols):
    x0 = _halves(jnp.concatenate([emb_mashup, emb_api], axis=0), pad_to=N_P)
    emm = _halves(emb_mm)
    eaa = _halves(emb_aa)

    gc, gr, gv = [], [], []
    for rows, cols, vals in ((graph_rows, graph_cols, graph_vals),
                             (diff1_rows, diff1_cols, diff1_vals),
                             (diff2_rows, diff2_cols, diff2_vals)):
        a, b_, v = _pad_edges(rows, cols, vals, EP_G)
        gc.append(a); gr.append(b_); gv.append(v)
    g_cols = jnp.stack(gc); g_rows = jnp.stack(gr); g_vals = jnp.stack(gv)
    mm_c, mm_r, mm_v = _pad_edges(mm_rows, mm_cols, mm_vals, EP_S)
    aa_c, aa_r, aa_v = _pad_edges(aa_rows, aa_cols, aa_vals, EP_S,
                                  row_off=N_M)

    midx = mashups.reshape(32, 1, 128).astype(jnp.int32)
    pidx = (N_M + pos_apis).reshape(32, 1, 128).astype(jnp.int32)
    nidx = (N_M + neg_apis).reshape(32, 1, 128).astype(jnp.int32)

    gath = _sc_forward(x0, emm, eaa, g_cols, g_rows, g_vals,
                       mm_c, mm_r, mm_v, aa_c, aa_r, aa_v,
                       midx, pidx, nidx)[0]

    gathered = []
    for g in range(3):
        for kind in range(3):
            gathered.append(jnp.concatenate(
                [gath[0, g, kind], gath[1, g, kind]], axis=1))

    loss, reg, sslm, ssla = _loss_call(mashups, pos_apis, gathered)
    return (loss[0, 0], reg[0, 0], sslm[0, 0], ssla[0, 0])
